# baseline (speedup 1.0000x reference)
import functools

import jax
import jax.numpy as jnp
from jax import lax
from jax.experimental import pallas as pl
from jax.experimental.pallas import tpu as pltpu
from jax.experimental.pallas import tpu_sc as plsc

NUM_CLASSES = 1000
ROWS = 4096
COLS = 20
N = ROWS * COLS
L = 16
NW = 32
PER_W = N // NW
WORDS_W = PER_W * NUM_CLASSES
ZWORDS = 102_400
NZDMA = WORDS_W // ZWORDS
SCAT = 128
NSCAT = PER_W // SCAT


def _make_sc_one_hot():
    mesh = plsc.VectorSubcoreMesh(core_axis_name="c", subcore_axis_name="s")

    @functools.partial(
        pl.kernel,
        mesh=mesh,
        compiler_params=pltpu.CompilerParams(needs_layout_passes=False),
        out_type=jax.ShapeDtypeStruct((N * NUM_CLASSES,), jnp.float32),
        scratch_types=[
            pltpu.VMEM((ZWORDS,), jnp.float32),
            pltpu.VMEM((PER_W,), jnp.int32),
            pltpu.VMEM((NSCAT, SCAT), jnp.int32),
            pltpu.VMEM((NSCAT, SCAT), jnp.float32),
            pltpu.SemaphoreType.DMA,
            pltpu.SemaphoreType.DMA,
        ],
    )
    def k(x_hbm, out_hbm, zbuf, idx_v, off2, val2, sem_z, sem_s):
        wid = lax.axis_index("s") * 2 + lax.axis_index("c")
        row_base = wid * PER_W
        word_base = row_base * NUM_CLASSES

        pltpu.sync_copy(x_hbm.at[pl.ds(row_base, PER_W)], idx_v)

        z = jnp.zeros((L,), jnp.float32)

        def zbody(i, _):
            for u in range(8):
                zbuf[pl.ds((i * 8 + u) * L, L)] = z
            return 0

        lax.fori_loop(0, ZWORDS // (8 * L), zbody, 0)

        for d in range(NZDMA):
            pltpu.async_copy(
                zbuf, out_hbm.at[pl.ds(word_base + d * ZWORDS, ZWORDS)], sem_z
            )

        iota = lax.iota(jnp.int32, L)

        def pbody(j, _):
            for o in range(SCAT // L):
                r = j * SCAT + o * L
                ids = idx_v[pl.ds(r, L)]
                valid = (ids >= 0) & (ids < NUM_CLASSES)
                rows = row_base + r + iota
                off2[j, pl.ds(o * L, L)] = rows * NUM_CLASSES + jnp.where(
                    valid, ids, 0
                )
                val2[j, pl.ds(o * L, L)] = jnp.where(valid, 1.0, 0.0)
            return 0

        lax.fori_loop(0, NSCAT, pbody, 0)

        for d in range(NZDMA):
            pltpu.make_async_copy(
                zbuf, out_hbm.at[pl.ds(word_base + d * ZWORDS, ZWORDS)], sem_z
            ).wait()

        for j in range(NSCAT):
            pltpu.async_copy(val2.at[j], out_hbm.at[off2.at[j]], sem_s)
        for j in range(NSCAT):
            pltpu.make_async_copy(val2.at[j], out_hbm.at[off2.at[j]], sem_s).wait()

    return k


_sc_one_hot = _make_sc_one_hot()


@jax.jit
def kernel(x):
    xf = x.reshape(-1).astype(jnp.int32)
    out = _sc_one_hot(xf)
    return out.reshape(ROWS, COLS, NUM_CLASSES)


# speedup vs baseline: 1.1057x; 1.1057x over previous
import functools

import jax
import jax.numpy as jnp
from jax import lax
from jax.experimental import pallas as pl
from jax.experimental.pallas import tpu as pltpu
from jax.experimental.pallas import tpu_sc as plsc

NUM_CLASSES = 1000
ROWS = 4096
COLS = 20
L = 16
NW = 32
D0_W = ROWS // NW
D0_C = 2
NCHUNK = D0_W // D0_C
IDS_W = D0_W * COLS
IDS_C = D0_C * COLS
SLOT = 48


def _make_sc_one_hot():
    mesh = plsc.VectorSubcoreMesh(core_axis_name="c", subcore_axis_name="s")

    @functools.partial(
        pl.kernel,
        mesh=mesh,
        compiler_params=pltpu.CompilerParams(
            needs_layout_passes=False, use_tc_tiling_on_sc=False
        ),
        out_type=jax.ShapeDtypeStruct((ROWS, COLS, NUM_CLASSES), jnp.float32),
        scratch_types=[
            pltpu.VMEM((IDS_W,), jnp.int32),
            pltpu.VMEM((NCHUNK * SLOT,), jnp.int32),
            pltpu.VMEM((D0_C, COLS, NUM_CLASSES), jnp.float32),
            pltpu.VMEM((D0_C, COLS, NUM_CLASSES), jnp.float32),
            pltpu.SemaphoreType.DMA,
            pltpu.SemaphoreType.DMA,
        ],
    )
    def k(x_hbm, out3_hbm, idx_v, idxp, buf0, buf1, sem0, sem1):
        wid = lax.axis_index("s") * 2 + lax.axis_index("c")
        d0_base = wid * D0_W

        pltpu.sync_copy(x_hbm.at[pl.ds(wid * IDS_W, IDS_W)], idx_v)

        iota = lax.iota(jnp.int32, L)

        def pbody(g, _):
            t = g * L + iota
            ids = idx_v[pl.ds(g * L, L)]
            dest = (t // IDS_C) * SLOT + t % IDS_C
            plsc.store_scatter(idxp, [dest], ids)
            return 0

        lax.fori_loop(0, IDS_W // L, pbody, 0)

        z = jnp.zeros((L,), jnp.float32)

        def zbody(r, _):
            i = r // COLS
            j = r % COLS
            for buf in (buf0, buf1):
                for s in range(NUM_CLASSES // L):
                    buf[i, j, pl.ds(s * L, L)] = z
                buf[i, j, pl.ds(NUM_CLASSES - L, L)] = z
            return 0

        lax.fori_loop(0, D0_C * COLS, zbody, 0)

        def scatter(buf, c, value):
            vals = jnp.full((L,), value, jnp.float32)
            for g in range(SLOT // L):
                s = iota + g * L
                ids = idxp[pl.ds(c * SLOT + g * L, L)]
                valid = (ids >= 0) & (ids < NUM_CLASSES) & (s < IDS_C)
                i = s // COLS
                j = s % COLS
                plsc.store_scatter(
                    buf, [i, j, jnp.where(valid, ids, 0)], vals, mask=valid
                )

        bufs = (buf0, buf1)
        sems = (sem0, sem1)

        def fire(c, buf, sem):
            pltpu.async_copy(
                buf, out3_hbm.at[pl.ds(d0_base + c * D0_C, D0_C)], sem
            )

        def drain(c, buf, sem):
            pltpu.make_async_copy(
                buf, out3_hbm.at[pl.ds(d0_base + c * D0_C, D0_C)], sem
            ).wait()

        for b in range(2):
            scatter(bufs[b], b, 1.0)
            fire(b, bufs[b], sems[b])

        def body(g, _):
            for b in range(2):
                c = g + b
                drain(c - 2, bufs[b], sems[b])
                scatter(bufs[b], c - 2, 0.0)
                scatter(bufs[b], c, 1.0)
                fire(c, bufs[b], sems[b])
            return 0

        lax.fori_loop(1, NCHUNK // 2, lambda g, s: body(g * 2, s), 0)

        for b in range(2):
            drain(NCHUNK - 2 + b, bufs[b], sems[b])

    return k


_sc_one_hot = _make_sc_one_hot()


def kernel(x):
    xf = x.reshape(-1).astype(jnp.int32)
    return _sc_one_hot(xf)


# speedup vs baseline: 6.3612x; 5.7529x over previous
import functools

import jax
import jax.numpy as jnp
from jax import lax
from jax.experimental import pallas as pl
from jax.experimental.pallas import tpu as pltpu
from jax.experimental.pallas import tpu_sc as plsc

NUM_CLASSES = 1000
ROWS = 4096
COLS = 20
L = 16
NW = 32
SLAB = ROWS // NW
TRPC = 25
KPC = 8 * TRPC
CPJ = NUM_CLASSES // KPC
NCHUNK = COLS * CPJ
GRP = SLAB // L


def _make_sc_one_hot():
    mesh = plsc.VectorSubcoreMesh(core_axis_name="c", subcore_axis_name="s")

    @functools.partial(
        pl.kernel,
        mesh=mesh,
        compiler_params=pltpu.CompilerParams(needs_layout_passes=False),
        out_type=jax.ShapeDtypeStruct((COLS, NUM_CLASSES, ROWS), jnp.float32),
        scratch_types=[
            pltpu.VMEM((COLS, SLAB), jnp.int32),
            pltpu.VMEM((KPC, SLAB), jnp.float32),
            pltpu.VMEM((KPC, SLAB), jnp.float32),
            pltpu.SemaphoreType.DMA,
            pltpu.SemaphoreType.DMA,
        ],
    )
    def k(xt_hbm, out_hbm, idb, buf0, buf1, sem0, sem1):
        wid = lax.axis_index("s") * 2 + lax.axis_index("c")
        d0_base = wid * SLAB

        pltpu.sync_copy(xt_hbm.at[:, pl.ds(d0_base, SLAB)], idb)

        iota = lax.iota(jnp.int32, L)

        z = jnp.zeros((L,), jnp.float32)

        def zbody(i, _):
            for buf in (buf0, buf1):
                for s in range(SLAB // L):
                    buf[i, pl.ds(s * L, L)] = z
            return 0

        lax.fori_loop(0, KPC, zbody, 0)

        def scatter(buf, q, value):
            j = q // CPJ
            kbase = (q % CPJ) * KPC
            vals = jnp.full((L,), value, jnp.float32)
            for g in range(GRP):
                ids = idb[j, pl.ds(g * L, L)]
                r = ids - kbase
                match = (r >= 0) & (r < KPC)
                plsc.store_scatter(
                    buf, [jnp.where(match, r, 0), iota + g * L], vals, mask=match
                )

        bufs = (buf0, buf1)
        sems = (sem0, sem1)

        def dst(q):
            j = q // CPJ
            kbase = (q % CPJ) * KPC
            return out_hbm.at[j, pl.ds(kbase, KPC), pl.ds(d0_base, SLAB)]

        def fire(q, buf, sem):
            pltpu.async_copy(buf, dst(q), sem)

        def drain(q, buf, sem):
            pltpu.make_async_copy(buf, dst(q), sem).wait()

        for b in range(2):
            scatter(bufs[b], jnp.int32(b), 1.0)
            fire(jnp.int32(b), bufs[b], sems[b])

        def body(g, _):
            for b in range(2):
                q = g + b
                drain(q - 2, bufs[b], sems[b])
                scatter(bufs[b], q - 2, 0.0)
                scatter(bufs[b], q, 1.0)
                fire(q, bufs[b], sems[b])
            return 0

        lax.fori_loop(1, NCHUNK // 2, lambda g, s: body(g * 2, s), 0)

        for b in range(2):
            drain(jnp.int32(NCHUNK - 2 + b), bufs[b], sems[b])

    return k


_sc_one_hot = _make_sc_one_hot()


def kernel(x):
    xt = jnp.transpose(x.astype(jnp.int32))
    out_t = _sc_one_hot(xt)
    return jnp.transpose(out_t, (2, 0, 1))


# speedup vs baseline: 7.4884x; 1.1772x over previous
import functools

import jax
import jax.numpy as jnp
from jax import lax
from jax.experimental import pallas as pl
from jax.experimental.pallas import tpu as pltpu
from jax.experimental.pallas import tpu_sc as plsc

NUM_CLASSES = 1000
ROWS = 4096
COLS = 20
L = 16
NW = 32
SLAB = ROWS // NW
TRPC = 50
KPC = 8 * TRPC
CPJ = NUM_CLASSES // KPC
NCHUNK = COLS * CPJ
GRP = SLAB // L


def _make_sc_one_hot():
    mesh = plsc.VectorSubcoreMesh(core_axis_name="c", subcore_axis_name="s")

    @functools.partial(
        pl.kernel,
        mesh=mesh,
        compiler_params=pltpu.CompilerParams(needs_layout_passes=False),
        out_type=jax.ShapeDtypeStruct((COLS, NUM_CLASSES, ROWS), jnp.float32),
        scratch_types=[
            pltpu.VMEM((COLS, SLAB), jnp.int32),
            pltpu.VMEM((KPC, SLAB), jnp.float32),
            pltpu.VMEM((KPC, SLAB), jnp.float32),
            pltpu.SemaphoreType.DMA,
            pltpu.SemaphoreType.DMA,
        ],
    )
    def k(xt_hbm, out_hbm, idb, buf0, buf1, sem0, sem1):
        wid = lax.axis_index("s") * 2 + lax.axis_index("c")
        d0_base = wid * SLAB

        pltpu.sync_copy(xt_hbm.at[:, pl.ds(d0_base, SLAB)], idb)

        iota = lax.iota(jnp.int32, L)

        z = jnp.zeros((L,), jnp.float32)

        def zbody(i, _):
            for buf in (buf0, buf1):
                for s in range(SLAB // L):
                    buf[i, pl.ds(s * L, L)] = z
            return 0

        lax.fori_loop(0, KPC, zbody, 0)

        def scatter(buf, q, value):
            j = q // CPJ
            kbase = (q % CPJ) * KPC
            vals = jnp.full((L,), value, jnp.float32)
            for g in range(GRP):
                ids = idb[j, pl.ds(g * L, L)]
                r = ids - kbase
                match = (r >= 0) & (r < KPC)
                plsc.store_scatter(
                    buf, [jnp.where(match, r, 0), iota + g * L], vals, mask=match
                )

        bufs = (buf0, buf1)
        sems = (sem0, sem1)

        def dst(q):
            j = q // CPJ
            kbase = (q % CPJ) * KPC
            return out_hbm.at[j, pl.ds(kbase, KPC), pl.ds(d0_base, SLAB)]

        def fire(q, buf, sem):
            pltpu.async_copy(buf, dst(q), sem)

        def drain(q, buf, sem):
            pltpu.make_async_copy(buf, dst(q), sem).wait()

        for b in range(2):
            scatter(bufs[b], jnp.int32(b), 1.0)
            fire(jnp.int32(b), bufs[b], sems[b])

        def body(g, _):
            for b in range(2):
                q = g + b
                drain(q - 2, bufs[b], sems[b])
                scatter(bufs[b], q - 2, 0.0)
                scatter(bufs[b], q, 1.0)
                fire(q, bufs[b], sems[b])
            return 0

        lax.fori_loop(1, NCHUNK // 2, lambda g, s: body(g * 2, s), 0)

        for b in range(2):
            drain(jnp.int32(NCHUNK - 2 + b), bufs[b], sems[b])

    return k


_sc_one_hot = _make_sc_one_hot()


def kernel(x):
    xt = jnp.transpose(x.astype(jnp.int32))
    out_t = _sc_one_hot(xt)
    return jnp.transpose(out_t, (2, 0, 1))
